# baseline (speedup 1.0000x reference)
import functools

import jax
import jax.numpy as jnp
from jax import lax
from jax.experimental import pallas as pl
from jax.experimental.pallas import tpu as pltpu
from jax.experimental.pallas import tpu_sc as plsc

G = 4
B = 2500
L = 16
D = 256
A = 512
BPAD = 2560
N = G * BPAD
NC, NS = 2, 16
NW = NC * NS
TILE_ROWS = N // NW
CHUNKS = (128, 128, 64)
NBUF = 7
NCH = 5
CH = TILE_ROWS // NCH
IDX_PER_TILE = 2 * L * TILE_ROWS
BM = 512
NBLK = N // BM

@functools.cache
def _sc_kernels(nslots):
    mesh = plsc.VectorSubcoreMesh(core_axis_name="c", subcore_axis_name="s",
                                  num_cores=NC, num_subcores=NS)
    tile_rows = nslots // NW
    nch = NCH if tile_rows % (NCH * 64) == 0 else 2
    ch = tile_rows // nch
    nbuf = NBUF if tile_rows % (NCH * 64) == 0 else 5
    idx_per_tile = 2 * L * tile_rows

    @functools.partial(
        pl.kernel,
        out_type=(
            jax.ShapeDtypeStruct((L, nslots, D), jnp.float32),
            jax.ShapeDtypeStruct((L, nslots, D), jnp.float32),
        ),
        mesh=mesh,
        scratch_types=[
            pltpu.VMEM((idx_per_tile,), jnp.int32),
            pltpu.VMEM((nbuf, ch, D), jnp.float32),
            pltpu.SemaphoreType.DMA((nbuf,)),
            pltpu.SemaphoreType.DMA((nbuf,)),
        ],
    )
    def sc_gather(idx_hbm, tdx_hbm, tan_hbm,
                  leaf_out, comb_out, idx_v, rowbuf, gsem, wsem):
        wid = lax.axis_index("s") * NC + lax.axis_index("c")
        tile_base = wid * tile_rows
        ibase = pl.multiple_of(wid * idx_per_tile, 64)
        pltpu.sync_copy(idx_hbm.at[pl.ds(ibase, idx_per_tile)], idx_v)

        @pl.loop(0, L)
        def _(l):
            off = l * (2 * tile_rows)
            units = ([(tdx_hbm, leaf_out, c) for c in range(nch)]
                     + [(tan_hbm, comb_out, c) for c in range(nch)])
            w = [None] * nbuf
            prev = None
            for u, (table, dst, c) in enumerate(units):
                b = u % nbuf
                if w[b] is not None:
                    w[b].wait()
                o = pl.multiple_of(off + u * ch, 8)
                gd = pltpu.async_copy(
                    table.at[idx_v.at[pl.ds(o, ch)]], rowbuf.at[b], gsem.at[b])
                if prev is not None:
                    pb, pd, pdst, pc = prev
                    pd.wait()
                    w[pb] = pltpu.async_copy(
                        rowbuf.at[pb],
                        pdst.at[l, pl.ds(tile_base + pc * ch, ch)],
                        wsem.at[pb])
                prev = (b, gd, dst, c)
            pb, pd, pdst, pc = prev
            pd.wait()
            w[pb] = pltpu.async_copy(
                rowbuf.at[pb], pdst.at[l, pl.ds(tile_base + pc * ch, ch)],
                wsem.at[pb])
            for wd in w:
                if wd is not None:
                    wd.wait()

    @functools.partial(
        pl.kernel,
        out_type=jax.ShapeDtypeStruct((N, D), jnp.float32),
        mesh=mesh,
        scratch_types=[
            pltpu.VMEM((max(CHUNKS),), jnp.int32),
            pltpu.VMEM((max(CHUNKS), D), jnp.float32),
            pltpu.SemaphoreType.DMA,
        ],
    )
    def sc_permute(idx_hbm, emb_hbm, out_hbm, idx_v, rows_v, sem):
        wid = lax.axis_index("s") * NC + lax.axis_index("c")
        off = 0
        for cn in CHUNKS:
            base = wid * TILE_ROWS + off
            pltpu.sync_copy(idx_hbm.at[pl.ds(base, cn)], idx_v.at[pl.ds(0, cn)])
            pltpu.async_copy(emb_hbm.at[idx_v.at[pl.ds(0, cn)]],
                             rows_v.at[pl.ds(0, cn)], sem).wait()
            pltpu.sync_copy(rows_v.at[pl.ds(0, cn)], out_hbm.at[pl.ds(base, cn)])
            off += cn

    return sc_gather, sc_permute


NRPAD = 128


def _tc_attend(leaf_ref, comb_ref, rel_ref, tre_ref, w1_ref, w2_ref, w12_ref,
               b_ref, cw_ref, t_ref, out_ref):
    pres = []
    combs = []
    rel_iota = lax.broadcasted_iota(jnp.int32, (BM, NRPAD), 1)
    tb = jnp.dot(t_ref[...], w2_ref[...],
                 preferred_element_type=jnp.float32)
    for l in range(L + 1):
        if l < L:
            lf = leaf_ref[l]
            oh = (rel_ref[l][:, None] == rel_iota).astype(jnp.float32)
            cb = comb_ref[l] + jnp.dot(oh, tre_ref[...],
                                       preferred_element_type=jnp.float32)
            x = jnp.dot(lf, w1_ref[...], preferred_element_type=jnp.float32)
            x = x + jnp.dot(cb, w2_ref[...],
                            preferred_element_type=jnp.float32)
        else:
            lf = leaf_ref[0]
            cb = lf + t_ref[...]
            x = jnp.dot(lf, w12_ref[...],
                        preferred_element_type=jnp.float32) + tb
        combs.append(cb)
        x = jnp.tanh(x + b_ref[...])
        pres.append(jnp.sum(x * cw_ref[...], axis=1, keepdims=True))
    p = jnp.concatenate(pres, axis=1)
    m = jnp.max(p, axis=1, keepdims=True)
    e = jnp.exp(p - m)
    s = jnp.sum(e, axis=1, keepdims=True)
    acc = combs[0] * (e[:, 0:1] / s)
    for l in range(1, L + 1):
        acc = acc + combs[l] * (e[:, l:l + 1] / s)
    out_ref[...] = acc


NSPLIT = 4
NH = N // NSPLIT


def kernel(dxEmb, leavesList, ancestorsList, relationList, permute_index,
           table_dx, table_t, table_an, table_re, attn_w, attn_b, comb_w,
           comb_b):
    del dxEmb, comb_b
    def prep(idx):
        idx = jnp.pad(idx.astype(jnp.int32), ((0, 0), (0, BPAD - B), (0, 0)))
        return idx.transpose(2, 0, 1).reshape(L, N)

    il2 = prep(leavesList)
    ia2 = prep(ancestorsList)
    ib2 = prep(relationList)

    sc_gather = _sc_kernels(NH)[0]
    sc_permute = _sc_kernels(N)[1]
    tre_pad = jnp.pad(table_re, ((0, NRPAD - (table_re.shape[0])), (0, 0)))
    w1 = attn_w[:D]
    w2 = attn_w[D:]
    w12 = w1 + w2

    gathered = []
    for h in range(NSPLIT):
        sl = slice(h * NH, (h + 1) * NH)
        il3 = il2[:, sl].reshape(L, NW, NH // NW)
        ia3 = ia2[:, sl].reshape(L, NW, NH // NW)
        pairs = jnp.stack([il3, ia3], axis=1)
        all_idx = pairs.transpose(2, 0, 1, 3).reshape(-1)
        gathered.append(sc_gather(all_idx, table_dx, table_an))

    outs = []
    for h in range(NSPLIT):
        sl = slice(h * NH, (h + 1) * NH)
        leaf_buf, comb_buf = gathered[h]
        out_h = pl.pallas_call(
            _tc_attend,
            grid=(NH // BM,),
            in_specs=[
                pl.BlockSpec((L, BM, D), lambda i: (0, i, 0)),
                pl.BlockSpec((L, BM, D), lambda i: (0, i, 0)),
                pl.BlockSpec((L, BM), lambda i: (0, i)),
                pl.BlockSpec((NRPAD, D), lambda i: (0, 0)),
                pl.BlockSpec((D, A), lambda i: (0, 0)),
                pl.BlockSpec((D, A), lambda i: (0, 0)),
                pl.BlockSpec((D, A), lambda i: (0, 0)),
                pl.BlockSpec((1, A), lambda i: (0, 0)),
                pl.BlockSpec((1, A), lambda i: (0, 0)),
                pl.BlockSpec((1, D), lambda i: (0, 0)),
            ],
            out_specs=pl.BlockSpec((BM, D), lambda i: (i, 0)),
            out_shape=jax.ShapeDtypeStruct((NH, D), jnp.float32),
        )(leaf_buf, comb_buf, ib2[:, sl], tre_pad, w1, w2, w12,
          attn_b.reshape(1, A), comb_w.reshape(1, A), table_t)
        outs.append(out_h)

    allEmb_p = jnp.concatenate(
        outs + [jnp.zeros((8, D), jnp.float32)], axis=0)
    p = permute_index.astype(jnp.int32)
    mapped = jnp.where(p == G * B, N, (p // B) * BPAD + p % B)
    mapped = jnp.concatenate(
        [mapped, jnp.zeros((N - (G * B + 1),), jnp.int32)])
    out = sc_permute(mapped, allEmb_p)
    return out[:G * B + 1]


# speedup vs baseline: 1.1861x; 1.1861x over previous
import functools

import jax
import jax.numpy as jnp
from jax import lax
from jax.experimental import pallas as pl
from jax.experimental.pallas import tpu as pltpu
from jax.experimental.pallas import tpu_sc as plsc

G = 4
B = 2500
L = 16
D = 256
A = 512
BPAD = 2560
N = G * BPAD
NC, NS = 2, 16
NW = NC * NS
TILE_ROWS = N // NW
CHUNKS = (128, 128, 64)
NBUF = 7
NCH = 5
CH = TILE_ROWS // NCH
IDX_PER_TILE = 2 * L * TILE_ROWS
BM = 512
NBLK = N // BM

@functools.cache
def _sc_kernels(nslots):
    mesh = plsc.VectorSubcoreMesh(core_axis_name="c", subcore_axis_name="s",
                                  num_cores=NC, num_subcores=NS)
    tile_rows = nslots // NW
    nch = NCH if tile_rows % (NCH * 64) == 0 else 2
    ch = tile_rows // nch
    nbuf = NBUF if tile_rows % (NCH * 64) == 0 else 5
    idx_per_tile = 2 * L * tile_rows

    @functools.partial(
        pl.kernel,
        out_type=(
            jax.ShapeDtypeStruct((L, nslots, D), jnp.float32),
            jax.ShapeDtypeStruct((L, nslots, D), jnp.float32),
        ),
        mesh=mesh,
        scratch_types=[
            pltpu.VMEM((idx_per_tile,), jnp.int32),
            pltpu.VMEM((nbuf, ch, D), jnp.float32),
            pltpu.SemaphoreType.DMA((nbuf,)),
            pltpu.SemaphoreType.DMA((nbuf,)),
        ],
    )
    def sc_gather(idx_hbm, tdx_hbm, tan_hbm,
                  leaf_out, comb_out, idx_v, rowbuf, gsem, wsem):
        wid = lax.axis_index("s") * NC + lax.axis_index("c")
        tile_base = wid * tile_rows
        ibase = pl.multiple_of(wid * idx_per_tile, 64)
        pltpu.sync_copy(idx_hbm.at[pl.ds(ibase, idx_per_tile)], idx_v)

        @pl.loop(0, L)
        def _(l):
            off = l * (2 * tile_rows)
            units = ([(tdx_hbm, leaf_out, c) for c in range(nch)]
                     + [(tan_hbm, comb_out, c) for c in range(nch)])
            w = [None] * nbuf
            prev = None
            for u, (table, dst, c) in enumerate(units):
                b = u % nbuf
                if w[b] is not None:
                    w[b].wait()
                o = pl.multiple_of(off + u * ch, 8)
                gd = pltpu.async_copy(
                    table.at[idx_v.at[pl.ds(o, ch)]], rowbuf.at[b], gsem.at[b])
                if prev is not None:
                    pb, pd, pdst, pc = prev
                    pd.wait()
                    w[pb] = pltpu.async_copy(
                        rowbuf.at[pb],
                        pdst.at[l, pl.ds(tile_base + pc * ch, ch)],
                        wsem.at[pb])
                prev = (b, gd, dst, c)
            pb, pd, pdst, pc = prev
            pd.wait()
            w[pb] = pltpu.async_copy(
                rowbuf.at[pb], pdst.at[l, pl.ds(tile_base + pc * ch, ch)],
                wsem.at[pb])
            for wd in w:
                if wd is not None:
                    wd.wait()

    @functools.partial(
        pl.kernel,
        out_type=jax.ShapeDtypeStruct((N, D), jnp.float32),
        mesh=mesh,
        scratch_types=[
            pltpu.VMEM((max(CHUNKS),), jnp.int32),
            pltpu.VMEM((max(CHUNKS), D), jnp.float32),
            pltpu.SemaphoreType.DMA,
        ],
    )
    def sc_permute(idx_hbm, emb_hbm, out_hbm, idx_v, rows_v, sem):
        wid = lax.axis_index("s") * NC + lax.axis_index("c")
        off = 0
        for cn in CHUNKS:
            base = wid * TILE_ROWS + off
            pltpu.sync_copy(idx_hbm.at[pl.ds(base, cn)], idx_v.at[pl.ds(0, cn)])
            pltpu.async_copy(emb_hbm.at[idx_v.at[pl.ds(0, cn)]],
                             rows_v.at[pl.ds(0, cn)], sem).wait()
            pltpu.sync_copy(rows_v.at[pl.ds(0, cn)], out_hbm.at[pl.ds(base, cn)])
            off += cn

    return sc_gather, sc_permute


NRPAD = 128


def _tc_attend(leaf_ref, comb_ref, rel_ref, tre_ref, w1_ref, w2_ref, w12_ref,
               b_ref, cw_ref, t_ref, out_ref):
    pres = []
    combs = []
    rel_iota = lax.broadcasted_iota(jnp.int32, (BM, NRPAD), 1)
    tb = jnp.dot(t_ref[...], w2_ref[...],
                 preferred_element_type=jnp.float32)
    for l in range(L + 1):
        if l < L:
            lf = leaf_ref[l]
            oh = (rel_ref[l][:, None] == rel_iota).astype(jnp.float32)
            cb = comb_ref[l] + jnp.dot(oh, tre_ref[...],
                                       preferred_element_type=jnp.float32)
            x = jnp.dot(lf, w1_ref[...], preferred_element_type=jnp.float32)
            x = x + jnp.dot(cb, w2_ref[...],
                            preferred_element_type=jnp.float32)
        else:
            lf = leaf_ref[0]
            cb = lf + t_ref[...]
            x = jnp.dot(lf, w12_ref[...],
                        preferred_element_type=jnp.float32) + tb
        combs.append(cb)
        x = jnp.tanh(x + b_ref[...])
        pres.append(jnp.sum(x * cw_ref[...], axis=1, keepdims=True))
    p = jnp.concatenate(pres, axis=1)
    m = jnp.max(p, axis=1, keepdims=True)
    e = jnp.exp(p - m)
    s = jnp.sum(e, axis=1, keepdims=True)
    acc = combs[0] * (e[:, 0:1] / s)
    for l in range(1, L + 1):
        acc = acc + combs[l] * (e[:, l:l + 1] / s)
    out_ref[...] = acc


NSPLIT = 2
NH = N // NSPLIT


def kernel(dxEmb, leavesList, ancestorsList, relationList, permute_index,
           table_dx, table_t, table_an, table_re, attn_w, attn_b, comb_w,
           comb_b):
    del dxEmb, comb_b
    def prep(idx):
        idx = jnp.pad(idx.astype(jnp.int32), ((0, 0), (0, BPAD - B), (0, 0)))
        return idx.transpose(2, 0, 1).reshape(L, N)

    il2 = prep(leavesList)
    ia2 = prep(ancestorsList)
    ib2 = prep(relationList)

    sc_gather = _sc_kernels(NH)[0]
    sc_permute = _sc_kernels(N)[1]
    tre_pad = jnp.pad(table_re, ((0, NRPAD - (table_re.shape[0])), (0, 0)))
    w1 = attn_w[:D]
    w2 = attn_w[D:]
    w12 = w1 + w2

    gathered = []
    for h in range(NSPLIT):
        sl = slice(h * NH, (h + 1) * NH)
        il3 = il2[:, sl].reshape(L, NW, NH // NW)
        ia3 = ia2[:, sl].reshape(L, NW, NH // NW)
        pairs = jnp.stack([il3, ia3], axis=1)
        all_idx = pairs.transpose(2, 0, 1, 3).reshape(-1)
        gathered.append(sc_gather(all_idx, table_dx, table_an))

    outs = []
    for h in range(NSPLIT):
        sl = slice(h * NH, (h + 1) * NH)
        leaf_buf, comb_buf = gathered[h]
        out_h = pl.pallas_call(
            _tc_attend,
            grid=(NH // BM,),
            in_specs=[
                pl.BlockSpec((L, BM, D), lambda i: (0, i, 0)),
                pl.BlockSpec((L, BM, D), lambda i: (0, i, 0)),
                pl.BlockSpec((L, BM), lambda i: (0, i)),
                pl.BlockSpec((NRPAD, D), lambda i: (0, 0)),
                pl.BlockSpec((D, A), lambda i: (0, 0)),
                pl.BlockSpec((D, A), lambda i: (0, 0)),
                pl.BlockSpec((D, A), lambda i: (0, 0)),
                pl.BlockSpec((1, A), lambda i: (0, 0)),
                pl.BlockSpec((1, A), lambda i: (0, 0)),
                pl.BlockSpec((1, D), lambda i: (0, 0)),
            ],
            out_specs=pl.BlockSpec((BM, D), lambda i: (i, 0)),
            out_shape=jax.ShapeDtypeStruct((NH, D), jnp.float32),
        )(leaf_buf, comb_buf, ib2[:, sl], tre_pad, w1, w2, w12,
          attn_b.reshape(1, A), comb_w.reshape(1, A), table_t)
        outs.append(out_h)

    allEmb_p = jnp.concatenate(
        outs + [jnp.zeros((8, D), jnp.float32)], axis=0)
    p = permute_index.astype(jnp.int32)
    mapped = jnp.where(p == G * B, N, (p // B) * BPAD + p % B)
    mapped = jnp.concatenate(
        [mapped, jnp.zeros((N - (G * B + 1),), jnp.int32)])
    out = sc_permute(mapped, allEmb_p)
    return out[:G * B + 1]


# speedup vs baseline: 1.3105x; 1.1049x over previous
import functools

import jax
import jax.numpy as jnp
from jax import lax
from jax.experimental import pallas as pl
from jax.experimental.pallas import tpu as pltpu
from jax.experimental.pallas import tpu_sc as plsc

G = 4
B = 2500
L = 16
D = 256
A = 512
BPAD = 2560
N = G * BPAD
NC, NS = 2, 16
NW = NC * NS
TILE_ROWS = N // NW
CHUNKS = (128, 128, 64)
NBUF = 7
NCH = 5
CH = TILE_ROWS // NCH
IDX_PER_TILE = 2 * L * TILE_ROWS
BM = 512
NBLK = N // BM

@functools.cache
def _sc_kernels(nslots):
    mesh = plsc.VectorSubcoreMesh(core_axis_name="c", subcore_axis_name="s",
                                  num_cores=NC, num_subcores=NS)
    tile_rows = nslots // NW
    ch = 80 if tile_rows % 80 == 0 else 64
    nch = tile_rows // ch
    idx_per_tile = 2 * L * tile_rows
    nbuf = min(7, max(3, (460 * 1024 - idx_per_tile * 4) // (ch * D * 4)))

    @functools.partial(
        pl.kernel,
        out_type=(
            jax.ShapeDtypeStruct((L, nslots, D), jnp.float32),
            jax.ShapeDtypeStruct((L, nslots, D), jnp.float32),
        ),
        mesh=mesh,
        scratch_types=[
            pltpu.VMEM((idx_per_tile,), jnp.int32),
            pltpu.VMEM((nbuf, ch, D), jnp.float32),
            pltpu.SemaphoreType.DMA((nbuf,)),
            pltpu.SemaphoreType.DMA((nbuf,)),
        ],
    )
    def sc_gather(idx_hbm, tdx_hbm, tan_hbm,
                  leaf_out, comb_out, idx_v, rowbuf, gsem, wsem):
        wid = lax.axis_index("s") * NC + lax.axis_index("c")
        tile_base = wid * tile_rows
        ibase = pl.multiple_of(wid * idx_per_tile, 64)
        pltpu.sync_copy(idx_hbm.at[pl.ds(ibase, idx_per_tile)], idx_v)

        @pl.loop(0, L)
        def _(l):
            off = l * (2 * tile_rows)
            units = ([(tdx_hbm, leaf_out, c) for c in range(nch)]
                     + [(tan_hbm, comb_out, c) for c in range(nch)])
            w = [None] * nbuf
            prev = None
            for u, (table, dst, c) in enumerate(units):
                b = u % nbuf
                if w[b] is not None:
                    w[b].wait()
                o = pl.multiple_of(off + u * ch, 8)
                gd = pltpu.async_copy(
                    table.at[idx_v.at[pl.ds(o, ch)]], rowbuf.at[b], gsem.at[b])
                if prev is not None:
                    pb, pd, pdst, pc = prev
                    pd.wait()
                    w[pb] = pltpu.async_copy(
                        rowbuf.at[pb],
                        pdst.at[l, pl.ds(tile_base + pc * ch, ch)],
                        wsem.at[pb])
                prev = (b, gd, dst, c)
            pb, pd, pdst, pc = prev
            pd.wait()
            w[pb] = pltpu.async_copy(
                rowbuf.at[pb], pdst.at[l, pl.ds(tile_base + pc * ch, ch)],
                wsem.at[pb])
            for wd in w:
                if wd is not None:
                    wd.wait()

    @functools.partial(
        pl.kernel,
        out_type=jax.ShapeDtypeStruct((N, D), jnp.float32),
        mesh=mesh,
        scratch_types=[
            pltpu.VMEM((max(CHUNKS),), jnp.int32),
            pltpu.VMEM((max(CHUNKS), D), jnp.float32),
            pltpu.SemaphoreType.DMA,
        ],
    )
    def sc_permute(idx_hbm, emb_hbm, out_hbm, idx_v, rows_v, sem):
        wid = lax.axis_index("s") * NC + lax.axis_index("c")
        off = 0
        for cn in CHUNKS:
            base = wid * TILE_ROWS + off
            pltpu.sync_copy(idx_hbm.at[pl.ds(base, cn)], idx_v.at[pl.ds(0, cn)])
            pltpu.async_copy(emb_hbm.at[idx_v.at[pl.ds(0, cn)]],
                             rows_v.at[pl.ds(0, cn)], sem).wait()
            pltpu.sync_copy(rows_v.at[pl.ds(0, cn)], out_hbm.at[pl.ds(base, cn)])
            off += cn

    return sc_gather, sc_permute


NRPAD = 128


def _tc_attend(leaf_ref, comb_ref, rel_ref, tre_ref, w1_ref, w2_ref, w12_ref,
               b_ref, cw_ref, t_ref, out_ref):
    pres = []
    combs = []
    rel_iota = lax.broadcasted_iota(jnp.int32, (BM, NRPAD), 1)
    tb = jnp.dot(t_ref[...], w2_ref[...],
                 preferred_element_type=jnp.float32)
    for l in range(L + 1):
        if l < L:
            lf = leaf_ref[l]
            oh = (rel_ref[l][:, None] == rel_iota).astype(jnp.float32)
            cb = comb_ref[l] + jnp.dot(oh, tre_ref[...],
                                       preferred_element_type=jnp.float32)
            x = jnp.dot(lf, w1_ref[...], preferred_element_type=jnp.float32)
            x = x + jnp.dot(cb, w2_ref[...],
                            preferred_element_type=jnp.float32)
        else:
            lf = leaf_ref[0]
            cb = lf + t_ref[...]
            x = jnp.dot(lf, w12_ref[...],
                        preferred_element_type=jnp.float32) + tb
        combs.append(cb)
        x = jnp.tanh(x + b_ref[...])
        pres.append(jnp.sum(x * cw_ref[...], axis=1, keepdims=True))
    p = jnp.concatenate(pres, axis=1)
    m = jnp.max(p, axis=1, keepdims=True)
    e = jnp.exp(p - m)
    s = jnp.sum(e, axis=1, keepdims=True)
    acc = combs[0] * (e[:, 0:1] / s)
    for l in range(1, L + 1):
        acc = acc + combs[l] * (e[:, l:l + 1] / s)
    out_ref[...] = acc


SPLITS = (7680, 2560)


def kernel(dxEmb, leavesList, ancestorsList, relationList, permute_index,
           table_dx, table_t, table_an, table_re, attn_w, attn_b, comb_w,
           comb_b):
    del dxEmb, comb_b
    def prep(idx):
        idx = jnp.pad(idx.astype(jnp.int32), ((0, 0), (0, BPAD - B), (0, 0)))
        return idx.transpose(2, 0, 1).reshape(L, N)

    il2 = prep(leavesList)
    ia2 = prep(ancestorsList)
    ib2 = prep(relationList)

    sc_permute = _sc_kernels(N)[1]
    tre_pad = jnp.pad(table_re, ((0, NRPAD - (table_re.shape[0])), (0, 0)))
    w1 = attn_w[:D]
    w2 = attn_w[D:]
    w12 = w1 + w2

    bounds = [0]
    for n in SPLITS:
        bounds.append(bounds[-1] + n)
    gathered = []
    for h, nh in enumerate(SPLITS):
        sl = slice(bounds[h], bounds[h + 1])
        il3 = il2[:, sl].reshape(L, NW, nh // NW)
        ia3 = ia2[:, sl].reshape(L, NW, nh // NW)
        pairs = jnp.stack([il3, ia3], axis=1)
        all_idx = pairs.transpose(2, 0, 1, 3).reshape(-1)
        gathered.append(_sc_kernels(nh)[0](all_idx, table_dx, table_an))

    outs = []
    for h, nh in enumerate(SPLITS):
        sl = slice(bounds[h], bounds[h + 1])
        leaf_buf, comb_buf = gathered[h]
        out_h = pl.pallas_call(
            _tc_attend,
            grid=(nh // BM,),
            in_specs=[
                pl.BlockSpec((L, BM, D), lambda i: (0, i, 0)),
                pl.BlockSpec((L, BM, D), lambda i: (0, i, 0)),
                pl.BlockSpec((L, BM), lambda i: (0, i)),
                pl.BlockSpec((NRPAD, D), lambda i: (0, 0)),
                pl.BlockSpec((D, A), lambda i: (0, 0)),
                pl.BlockSpec((D, A), lambda i: (0, 0)),
                pl.BlockSpec((D, A), lambda i: (0, 0)),
                pl.BlockSpec((1, A), lambda i: (0, 0)),
                pl.BlockSpec((1, A), lambda i: (0, 0)),
                pl.BlockSpec((1, D), lambda i: (0, 0)),
            ],
            out_specs=pl.BlockSpec((BM, D), lambda i: (i, 0)),
            out_shape=jax.ShapeDtypeStruct((nh, D), jnp.float32),
        )(leaf_buf, comb_buf, ib2[:, sl], tre_pad, w1, w2, w12,
          attn_b.reshape(1, A), comb_w.reshape(1, A), table_t)
        outs.append(out_h)

    allEmb_p = jnp.concatenate(
        outs + [jnp.zeros((8, D), jnp.float32)], axis=0)
    p = permute_index.astype(jnp.int32)
    mapped = jnp.where(p == G * B, N, (p // B) * BPAD + p % B)
    mapped = jnp.concatenate(
        [mapped, jnp.zeros((N - (G * B + 1),), jnp.int32)])
    out = sc_permute(mapped, allEmb_p)
    return out[:G * B + 1]
